# baseline (speedup 1.0000x reference)
import functools

import jax
import jax.numpy as jnp
from jax import lax
from jax.experimental import pallas as pl
from jax.experimental.pallas import tpu as pltpu
from jax.experimental.pallas import tpu_sc as plsc

_NC = 2
_NS = 16
_NW = _NC * _NS
_BB = 128
_L = 16


def _emb_call(S, B, D):
    assert B == _NW * _BB
    mesh = plsc.VectorSubcoreMesh(core_axis_name="c", subcore_axis_name="s")

    @functools.partial(
        pl.kernel,
        out_type=jax.ShapeDtypeStruct((S, D, B), jnp.float32),
        mesh=mesh,
        scratch_types=[
            pltpu.VMEM((_BB,), jnp.int32),
            pltpu.VMEM((_BB,), jnp.int32),
            pltpu.VMEM((_BB, 2 * D), jnp.float32),
            pltpu.VMEM((D, _BB), jnp.float32),
            pltpu.SemaphoreType.DMA,
            pltpu.SemaphoreType.DMA,
        ],
        compiler_params=pltpu.CompilerParams(needs_layout_passes=False),
    )
    def emb(idx_hbm, w2_hbm, out_hbm, idx_v, gidx_v, rows_v, out_v, sem_g,
            sem_w):
        wid = lax.axis_index("s") * _NC + lax.axis_index("c")
        col0 = wid * _BB

        def body(s, carry):
            pltpu.sync_copy(idx_hbm.at[s, pl.ds(col0, _BB)], idx_v)
            for g in range(_BB // _L):
                v = idx_v[pl.ds(g * _L, _L)]
                gidx_v[pl.ds(g * _L, _L)] = lax.shift_right_logical(v, 1)
            pltpu.async_copy(w2_hbm.at[gidx_v], rows_v, sem_g).wait()
            iota = lax.iota(jnp.int32, _L)
            for g in range(_BB // _L):
                v = idx_v[pl.ds(g * _L, _L)]
                pcol = lax.mul(lax.bitwise_and(v, 1), D)
                rvec = iota + g * _L
                for d in range(D):
                    x = plsc.load_gather(rows_v, [rvec, pcol + d])
                    out_v[d, pl.ds(g * _L, _L)] = x
            for r in range(D // 8):
                pltpu.async_copy(
                    out_v.at[pl.ds(r * 8, 8)],
                    out_hbm.at[s, pl.ds(r * 8, 8), pl.ds(col0, _BB)],
                    sem_w,
                )
            for r in range(D // 8):
                pltpu.make_async_copy(
                    out_v.at[pl.ds(r * 8, 8)],
                    out_hbm.at[s, pl.ds(r * 8, 8), pl.ds(col0, _BB)],
                    sem_w,
                ).wait()
            return carry

        lax.fori_loop(0, S, body, 0, unroll=False)

    return emb


def kernel(idx, weight):
    b, s = idx.shape
    v, d = weight.shape
    idx_t = idx.T.astype(jnp.int32)
    w2 = weight.reshape(v // 2, 2 * d)
    out_t = _emb_call(s, b, d)(idx_t, w2)
    return out_t.transpose(2, 0, 1)


# speedup vs baseline: 1.7207x; 1.7207x over previous
import functools

import jax
import jax.numpy as jnp
from jax import lax
from jax.experimental import pallas as pl
from jax.experimental.pallas import tpu as pltpu
from jax.experimental.pallas import tpu_sc as plsc

_NC = 2
_NS = 16
_NW = _NC * _NS
_BB = 128
_L = 16
_NG = _BB // _L


def _emb_call(S, B, D):
    assert B == _NW * _BB
    mesh = plsc.VectorSubcoreMesh(core_axis_name="c", subcore_axis_name="s")

    scratch = (
        [pltpu.VMEM((_BB,), jnp.int32) for _ in range(2)]
        + [pltpu.VMEM((_BB,), jnp.int32) for _ in range(2)]
        + [pltpu.VMEM((_BB,), jnp.int32) for _ in range(2)]
        + [pltpu.VMEM((_BB, 2 * D), jnp.float32) for _ in range(2)]
        + [pltpu.VMEM((D, _BB), jnp.float32) for _ in range(2)]
        + [pltpu.SemaphoreType.DMA for _ in range(6)]
    )

    @functools.partial(
        pl.kernel,
        out_type=jax.ShapeDtypeStruct((S, D, B), jnp.float32),
        mesh=mesh,
        scratch_types=scratch,
        compiler_params=pltpu.CompilerParams(needs_layout_passes=False),
    )
    def emb(idx_hbm, w2_hbm, out_hbm, *refs):
        idx_v = refs[0:2]
        gidx_v = refs[2:4]
        pcol_v = refs[4:6]
        rows_v = refs[6:8]
        out_v = refs[8:10]
        sem_i = refs[10:12]
        sem_g = refs[12:14]
        sem_w = refs[14:16]

        wid = lax.axis_index("s") * _NC + lax.axis_index("c")
        col0 = wid * _BB

        def load_idx(c, b, sync):
            cp = pltpu.async_copy(idx_hbm.at[c, pl.ds(col0, _BB)], idx_v[b],
                                  sem_i[b])
            if sync:
                cp.wait()

        def wait_idx(b):
            pltpu.make_async_copy(idx_hbm.at[0, pl.ds(col0, _BB)], idx_v[b],
                                  sem_i[b]).wait()

        def prep_idx(b):
            for g in range(_NG):
                v = idx_v[b][pl.ds(g * _L, _L)]
                gidx_v[b][pl.ds(g * _L, _L)] = lax.shift_right_logical(v, 1)
                pcol_v[b][pl.ds(g * _L, _L)] = lax.mul(
                    lax.bitwise_and(v, 1), D)

        def fire_gather(b):
            pltpu.async_copy(w2_hbm.at[gidx_v[b]], rows_v[b], sem_g[b])

        def wait_gather(b):
            pltpu.make_async_copy(w2_hbm.at[gidx_v[b]], rows_v[b],
                                  sem_g[b]).wait()

        def transpose(b):
            iota = lax.iota(jnp.int32, _L)
            rvecs = [iota + g * _L for g in range(_NG)]
            pcols = [pcol_v[b][pl.ds(g * _L, _L)] for g in range(_NG)]

            @plsc.parallel_loop(0, D, 1, unroll=4)
            def _(d):
                for g in range(_NG):
                    x = plsc.load_gather(rows_v[b], [rvecs[g], pcols[g] + d])
                    out_v[b][d, pl.ds(g * _L, _L)] = x

        def fire_writes(c, b):
            for r in range(D // 8):
                pltpu.async_copy(
                    out_v[b].at[pl.ds(r * 8, 8)],
                    out_hbm.at[c, pl.ds(r * 8, 8), pl.ds(col0, _BB)],
                    sem_w[b],
                )

        def wait_writes(b):
            for r in range(D // 8):
                pltpu.make_async_copy(
                    out_v[b].at[pl.ds(r * 8, 8)],
                    out_hbm.at[0, pl.ds(r * 8, 8), pl.ds(col0, _BB)],
                    sem_w[b],
                ).wait()

        load_idx(0, 0, True)
        prep_idx(0)
        fire_gather(0)
        load_idx(1, 1, False)

        def body(grp, carry):
            for b in range(2):
                c = grp * 2 + b
                nb = 1 - b
                if b == 0:
                    wait_idx(nb)
                    prep_idx(nb)

                    @pl.when(grp >= 1)
                    def _():
                        wait_writes(nb)
                    fire_gather(nb)

                    @pl.when(grp < (S // 2) - 1)
                    def _():
                        load_idx(c + 2, b, False)
                else:
                    @pl.when(grp < (S // 2) - 1)
                    def _():
                        wait_idx(nb)
                        prep_idx(nb)
                        wait_writes(nb)
                        fire_gather(nb)
                        load_idx(c + 2, b, False)
                wait_gather(b)
                transpose(b)
                fire_writes(c, b)
            return carry

        lax.fori_loop(0, S // 2, body, 0, unroll=False)
        wait_writes(0)
        wait_writes(1)

    return emb


def kernel(idx, weight):
    b, s = idx.shape
    v, d = weight.shape
    idx_t = idx.T.astype(jnp.int32)
    w2 = weight.reshape(v // 2, 2 * d)
    out_t = _emb_call(s, b, d)(idx_t, w2)
    return out_t.transpose(2, 0, 1)


# speedup vs baseline: 1.7246x; 1.0023x over previous
import functools

import jax
import jax.numpy as jnp
from jax import lax
from jax.experimental import pallas as pl
from jax.experimental.pallas import tpu as pltpu
from jax.experimental.pallas import tpu_sc as plsc

_NC = 2
_NS = 16
_NW = _NC * _NS
_BB = 128
_L = 16
_NG = _BB // _L


def _emb_call(S, B, D):
    assert B == _NW * _BB
    mesh = plsc.VectorSubcoreMesh(core_axis_name="c", subcore_axis_name="s")

    scratch = (
        [pltpu.VMEM((_BB,), jnp.int32) for _ in range(2)]
        + [pltpu.VMEM((_BB,), jnp.int32) for _ in range(2)]
        + [pltpu.VMEM((_BB,), jnp.int32) for _ in range(2)]
        + [pltpu.VMEM((_BB, 2 * D), jnp.float32) for _ in range(2)]
        + [pltpu.VMEM((D, _BB), jnp.float32) for _ in range(2)]
        + [pltpu.SemaphoreType.DMA for _ in range(6)]
    )

    @functools.partial(
        pl.kernel,
        out_type=jax.ShapeDtypeStruct((S, D, B), jnp.float32),
        mesh=mesh,
        scratch_types=scratch,
        compiler_params=pltpu.CompilerParams(needs_layout_passes=False),
    )
    def emb(idx_hbm, w2_hbm, out_hbm, *refs):
        idx_v = refs[0:2]
        gidx_v = refs[2:4]
        pcol_v = refs[4:6]
        rows_v = refs[6:8]
        out_v = refs[8:10]
        sem_i = refs[10:12]
        sem_g = refs[12:14]
        sem_w = refs[14:16]

        wid = lax.axis_index("s") * _NC + lax.axis_index("c")
        col0 = wid * _BB

        def load_idx(c, b, sync):
            cp = pltpu.async_copy(idx_hbm.at[c, pl.ds(col0, _BB)], idx_v[b],
                                  sem_i[b])
            if sync:
                cp.wait()

        def wait_idx(b):
            pltpu.make_async_copy(idx_hbm.at[0, pl.ds(col0, _BB)], idx_v[b],
                                  sem_i[b]).wait()

        def prep_idx(b):
            iota = lax.iota(jnp.int32, _L)
            for g in range(_NG):
                v = idx_v[b][pl.ds(g * _L, _L)]
                gidx_v[b][pl.ds(g * _L, _L)] = lax.shift_right_logical(v, 1)
                lane = iota + g * _L
                pcol_v[b][pl.ds(g * _L, _L)] = (
                    lax.shift_left(lane, 7)
                    + lax.mul(lax.bitwise_and(v, 1), D))

        def fire_gather(b):
            pltpu.async_copy(w2_hbm.at[gidx_v[b]], rows_v[b], sem_g[b])

        def wait_gather(b):
            pltpu.make_async_copy(w2_hbm.at[gidx_v[b]], rows_v[b],
                                  sem_g[b]).wait()

        def transpose(b):
            zero = jnp.zeros((_L,), jnp.int32)
            avecs = [pcol_v[b][pl.ds(g * _L, _L)] for g in range(_NG)]

            @plsc.parallel_loop(0, D, 1, unroll=8)
            def _(d):
                for g in range(_NG):
                    x = plsc.load_gather(rows_v[b], [zero, avecs[g] + d])
                    out_v[b][d, pl.ds(g * _L, _L)] = x

        def fire_writes(c, b):
            for r in range(D // 8):
                pltpu.async_copy(
                    out_v[b].at[pl.ds(r * 8, 8)],
                    out_hbm.at[c, pl.ds(r * 8, 8), pl.ds(col0, _BB)],
                    sem_w[b],
                )

        def wait_writes(b):
            for r in range(D // 8):
                pltpu.make_async_copy(
                    out_v[b].at[pl.ds(r * 8, 8)],
                    out_hbm.at[0, pl.ds(r * 8, 8), pl.ds(col0, _BB)],
                    sem_w[b],
                ).wait()

        load_idx(0, 0, True)
        prep_idx(0)
        fire_gather(0)
        load_idx(1, 1, False)

        def body(grp, carry):
            for b in range(2):
                c = grp * 2 + b
                nb = 1 - b
                if b == 0:
                    wait_idx(nb)
                    prep_idx(nb)

                    @pl.when(grp >= 1)
                    def _():
                        wait_writes(nb)
                    fire_gather(nb)

                    @pl.when(grp < (S // 2) - 1)
                    def _():
                        load_idx(c + 2, b, False)
                else:
                    @pl.when(grp < (S // 2) - 1)
                    def _():
                        wait_idx(nb)
                        prep_idx(nb)
                        wait_writes(nb)
                        fire_gather(nb)
                        load_idx(c + 2, b, False)
                wait_gather(b)
                transpose(b)
                fire_writes(c, b)
            return carry

        lax.fori_loop(0, S // 2, body, 0, unroll=False)
        wait_writes(0)
        wait_writes(1)

    return emb


def kernel(idx, weight):
    b, s = idx.shape
    v, d = weight.shape
    idx_t = idx.T.astype(jnp.int32)
    w2 = weight.reshape(v // 2, 2 * d)
    out_t = _emb_call(s, b, d)(idx_t, w2)
    return out_t.transpose(2, 0, 1)


# speedup vs baseline: 1.8032x; 1.0456x over previous
import functools

import jax
import jax.numpy as jnp
from jax import lax
from jax.experimental import pallas as pl
from jax.experimental.pallas import tpu as pltpu
from jax.experimental.pallas import tpu_sc as plsc

_NC = 2
_NS = 16
_NW = _NC * _NS
_BB = 128
_L = 16
_NG = _BB // _L


def _emb_call(S, B, D):
    assert B == _NW * _BB
    mesh = plsc.VectorSubcoreMesh(core_axis_name="c", subcore_axis_name="s")

    NB = 3
    scratch = (
        [pltpu.VMEM((_BB,), jnp.int32) for _ in range(NB)]
        + [pltpu.VMEM((_BB,), jnp.int32) for _ in range(NB)]
        + [pltpu.VMEM((_BB,), jnp.int32) for _ in range(NB)]
        + [pltpu.VMEM((_BB, 2 * D), jnp.float32) for _ in range(NB)]
        + [pltpu.VMEM((D, _BB), jnp.float32) for _ in range(NB)]
        + [pltpu.SemaphoreType.DMA for _ in range(3 * NB)]
    )

    @functools.partial(
        pl.kernel,
        out_type=jax.ShapeDtypeStruct((S, D, B), jnp.float32),
        mesh=mesh,
        scratch_types=scratch,
        compiler_params=pltpu.CompilerParams(needs_layout_passes=False),
    )
    def emb(idx_hbm, w2_hbm, out_hbm, *refs):
        idx_v = refs[0 * NB:1 * NB]
        gidx_v = refs[1 * NB:2 * NB]
        pcol_v = refs[2 * NB:3 * NB]
        rows_v = refs[3 * NB:4 * NB]
        out_v = refs[4 * NB:5 * NB]
        sem_i = refs[5 * NB:6 * NB]
        sem_g = refs[6 * NB:7 * NB]
        sem_w = refs[7 * NB:8 * NB]

        wid = lax.axis_index("s") * _NC + lax.axis_index("c")
        col0 = wid * _BB

        def load_idx(c, b, sync):
            cp = pltpu.async_copy(idx_hbm.at[c, pl.ds(col0, _BB)], idx_v[b],
                                  sem_i[b])
            if sync:
                cp.wait()

        def wait_idx(b):
            pltpu.make_async_copy(idx_hbm.at[0, pl.ds(col0, _BB)], idx_v[b],
                                  sem_i[b]).wait()

        def prep_idx(b):
            iota = lax.iota(jnp.int32, _L)
            for g in range(_NG):
                v = idx_v[b][pl.ds(g * _L, _L)]
                gidx_v[b][pl.ds(g * _L, _L)] = lax.shift_right_logical(v, 1)
                lane = iota + g * _L
                pcol_v[b][pl.ds(g * _L, _L)] = (
                    lax.shift_left(lane, 7)
                    + lax.mul(lax.bitwise_and(v, 1), D))

        def fire_gather(b):
            pltpu.async_copy(w2_hbm.at[gidx_v[b]], rows_v[b], sem_g[b])

        def wait_gather(b):
            pltpu.make_async_copy(w2_hbm.at[gidx_v[b]], rows_v[b],
                                  sem_g[b]).wait()

        def transpose(b):
            zero = jnp.zeros((_L,), jnp.int32)
            avecs = [pcol_v[b][pl.ds(g * _L, _L)] for g in range(_NG)]

            @plsc.parallel_loop(0, D, 1, unroll=8)
            def _(d):
                for g in range(_NG):
                    x = plsc.load_gather(rows_v[b], [zero, avecs[g] + d])
                    out_v[b][d, pl.ds(g * _L, _L)] = x

        def fire_writes(c, b):
            for r in range(D // 8):
                pltpu.async_copy(
                    out_v[b].at[pl.ds(r * 8, 8)],
                    out_hbm.at[c, pl.ds(r * 8, 8), pl.ds(col0, _BB)],
                    sem_w[b],
                )

        def wait_writes(b):
            for r in range(D // 8):
                pltpu.make_async_copy(
                    out_v[b].at[pl.ds(r * 8, 8)],
                    out_hbm.at[0, pl.ds(r * 8, 8), pl.ds(col0, _BB)],
                    sem_w[b],
                ).wait()

        load_idx(0, 0, True)
        prep_idx(0)
        fire_gather(0)
        load_idx(1, 1, True)
        prep_idx(1)
        fire_gather(1)
        load_idx(2, 2, False)

        GRPS = (S - 2) // NB

        def step(c, b, grp):
            b2 = (b + 2) % NB
            if grp is not None:
                wait_idx(b2)
                prep_idx(b2)
                fire_gather(b2)
                if b == 2:
                    @pl.when(grp < GRPS - 1)
                    def _():
                        load_idx(c + 3, b, False)
                else:
                    load_idx(c + 3, b, False)
            wait_gather(b)
            if grp is None:
                wait_writes(b)
            else:
                @pl.when(grp >= 1)
                def _():
                    wait_writes(b)
            transpose(b)
            fire_writes(c, b)

        def body(grp, carry):
            for b in range(NB):
                step(grp * NB + b, b, grp)
            return carry

        lax.fori_loop(0, GRPS, body, 0, unroll=False)
        step(S - 2, (S - 2) % NB, None)
        step(S - 1, (S - 1) % NB, None)
        for b in range(NB):
            wait_writes(b)

    return emb


def kernel(idx, weight):
    b, s = idx.shape
    v, d = weight.shape
    idx_t = idx.T.astype(jnp.int32)
    w2 = weight.reshape(v // 2, 2 * d)
    out_t = _emb_call(s, b, d)(idx_t, w2)
    return out_t.transpose(2, 0, 1)


# speedup vs baseline: 2.7194x; 1.5081x over previous
import functools

import jax
import jax.numpy as jnp
from jax import lax
from jax.experimental import pallas as pl
from jax.experimental.pallas import tpu as pltpu
from jax.experimental.pallas import tpu_sc as plsc

_NC = 2
_NS = 16
_NW = _NC * _NS
_BB = 128
_L = 16
_NG = _BB // _L


def _emb_call(S, B, D):
    assert B == _NW * _BB
    mesh = plsc.VectorSubcoreMesh(core_axis_name="c", subcore_axis_name="s")

    NB = 3
    scratch = (
        [pltpu.VMEM((_BB,), jnp.int32) for _ in range(NB)]
        + [pltpu.VMEM((_BB,), jnp.int32) for _ in range(NB)]
        + [pltpu.VMEM((_BB,), jnp.int32) for _ in range(NB)]
        + [pltpu.VMEM((_BB, 2 * D), jnp.float32) for _ in range(NB)]
        + [pltpu.VMEM((D, _BB), jnp.float32) for _ in range(NB)]
        + [pltpu.SemaphoreType.DMA for _ in range(3 * NB)]
    )

    @functools.partial(
        pl.kernel,
        out_type=jax.ShapeDtypeStruct((S, D, B), jnp.float32),
        mesh=mesh,
        scratch_types=scratch,
        compiler_params=pltpu.CompilerParams(needs_layout_passes=False),
    )
    def emb(idx_hbm, w2_hbm, out_hbm, *refs):
        idx_v = refs[0 * NB:1 * NB]
        gidx_v = refs[1 * NB:2 * NB]
        pcol_v = refs[2 * NB:3 * NB]
        rows_v = refs[3 * NB:4 * NB]
        out_v = refs[4 * NB:5 * NB]
        sem_i = refs[5 * NB:6 * NB]
        sem_g = refs[6 * NB:7 * NB]
        sem_w = refs[7 * NB:8 * NB]

        wid = lax.axis_index("s") * _NC + lax.axis_index("c")
        col0 = wid * _BB

        def load_idx(c, b, sync):
            cp = pltpu.async_copy(idx_hbm.at[c, pl.ds(col0, _BB)], idx_v[b],
                                  sem_i[b])
            if sync:
                cp.wait()

        def wait_idx(b):
            pltpu.make_async_copy(idx_hbm.at[0, pl.ds(col0, _BB)], idx_v[b],
                                  sem_i[b]).wait()

        def prep_idx(b):
            iota = lax.iota(jnp.int32, _L)
            for g in range(_NG):
                v = idx_v[b][pl.ds(g * _L, _L)]
                gidx_v[b][pl.ds(g * _L, _L)] = lax.shift_right_logical(v, 1)
                lane = iota + g * _L
                pcol_v[b][pl.ds(g * _L, _L)] = (
                    lax.shift_left(lane, 7)
                    + lax.mul(lax.bitwise_and(v, 1), D))

        def fire_gather(b):
            pltpu.async_copy(w2_hbm.at[gidx_v[b]], rows_v[b], sem_g[b])

        def wait_gather(b):
            pltpu.make_async_copy(w2_hbm.at[gidx_v[b]], rows_v[b],
                                  sem_g[b]).wait()

        def transpose(b):
            zero = jnp.zeros((_L,), jnp.int32)
            iota = lax.iota(jnp.int32, _L)
            base = [pcol_v[b][pl.ds(g * _L, _L)] for g in range(_NG)]

            @plsc.parallel_loop(0, _L, 1, unroll=2)
            def _(j):
                rot = lax.bitwise_and(iota + j, _L - 1)
                for dblk in range(D // _L):
                    rv = rot + dblk * _L
                    sb = rv * _BB + iota
                    for g in range(_NG):
                        x = plsc.load_gather(rows_v[b], [zero, base[g] + rv])
                        plsc.store_scatter(out_v[b], [zero, sb + g * _L], x)

        def fire_writes(c, b):
            for r in range(D // 8):
                pltpu.async_copy(
                    out_v[b].at[pl.ds(r * 8, 8)],
                    out_hbm.at[c, pl.ds(r * 8, 8), pl.ds(col0, _BB)],
                    sem_w[b],
                )

        def wait_writes(b):
            for r in range(D // 8):
                pltpu.make_async_copy(
                    out_v[b].at[pl.ds(r * 8, 8)],
                    out_hbm.at[0, pl.ds(r * 8, 8), pl.ds(col0, _BB)],
                    sem_w[b],
                ).wait()

        load_idx(0, 0, True)
        prep_idx(0)
        fire_gather(0)
        load_idx(1, 1, True)
        prep_idx(1)
        fire_gather(1)
        load_idx(2, 2, False)

        GRPS = (S - 2) // NB

        def step(c, b, grp):
            b2 = (b + 2) % NB
            if grp is not None:
                wait_idx(b2)
                prep_idx(b2)
                fire_gather(b2)
                if b == 2:
                    @pl.when(grp < GRPS - 1)
                    def _():
                        load_idx(c + 3, b, False)
                else:
                    load_idx(c + 3, b, False)
            wait_gather(b)
            if grp is None:
                wait_writes(b)
            else:
                @pl.when(grp >= 1)
                def _():
                    wait_writes(b)
            transpose(b)
            fire_writes(c, b)

        def body(grp, carry):
            for b in range(NB):
                step(grp * NB + b, b, grp)
            return carry

        lax.fori_loop(0, GRPS, body, 0, unroll=False)
        step(S - 2, (S - 2) % NB, None)
        step(S - 1, (S - 1) % NB, None)
        for b in range(NB):
            wait_writes(b)

    return emb


def kernel(idx, weight):
    b, s = idx.shape
    v, d = weight.shape
    idx_t = idx.T.astype(jnp.int32)
    w2 = weight.reshape(v // 2, 2 * d)
    out_t = _emb_call(s, b, d)(idx_t, w2)
    return out_t.transpose(2, 0, 1)


# speedup vs baseline: 2.7241x; 1.0018x over previous
import functools

import jax
import jax.numpy as jnp
from jax import lax
from jax.experimental import pallas as pl
from jax.experimental.pallas import tpu as pltpu
from jax.experimental.pallas import tpu_sc as plsc

_NC = 2
_NS = 16
_NW = _NC * _NS
_BB = 128
_L = 16
_NG = _BB // _L


def _emb_call(S, B, D):
    assert B == _NW * _BB
    mesh = plsc.VectorSubcoreMesh(core_axis_name="c", subcore_axis_name="s")

    NB = 3
    scratch = (
        [pltpu.VMEM((_BB,), jnp.int32) for _ in range(NB)]
        + [pltpu.VMEM((_BB,), jnp.int32) for _ in range(NB)]
        + [pltpu.VMEM((_BB,), jnp.int32) for _ in range(NB)]
        + [pltpu.VMEM((_BB, 2 * D), jnp.float32) for _ in range(NB)]
        + [pltpu.VMEM((D, _BB), jnp.float32) for _ in range(NB)]
        + [pltpu.SemaphoreType.DMA for _ in range(3 * NB)]
    )

    @functools.partial(
        pl.kernel,
        out_type=jax.ShapeDtypeStruct((S, D, B), jnp.float32),
        mesh=mesh,
        scratch_types=scratch,
        compiler_params=pltpu.CompilerParams(needs_layout_passes=False),
    )
    def emb(idx_hbm, w2_hbm, out_hbm, *refs):
        idx_v = refs[0 * NB:1 * NB]
        gidx_v = refs[1 * NB:2 * NB]
        pcol_v = refs[2 * NB:3 * NB]
        rows_v = refs[3 * NB:4 * NB]
        out_v = refs[4 * NB:5 * NB]
        sem_i = refs[5 * NB:6 * NB]
        sem_g = refs[6 * NB:7 * NB]
        sem_w = refs[7 * NB:8 * NB]

        wid = lax.axis_index("s") * _NC + lax.axis_index("c")
        col0 = wid * _BB

        def load_idx(c, b, sync):
            cp = pltpu.async_copy(idx_hbm.at[c, pl.ds(col0, _BB)], idx_v[b],
                                  sem_i[b])
            if sync:
                cp.wait()

        def wait_idx(b):
            pltpu.make_async_copy(idx_hbm.at[0, pl.ds(col0, _BB)], idx_v[b],
                                  sem_i[b]).wait()

        def prep_idx(b):
            iota = lax.iota(jnp.int32, _L)
            for g in range(_NG):
                v = idx_v[b][pl.ds(g * _L, _L)]
                gidx_v[b][pl.ds(g * _L, _L)] = lax.shift_right_logical(v, 1)
                lane = iota + g * _L
                pcol_v[b][pl.ds(g * _L, _L)] = (
                    lax.shift_left(lane, 7)
                    + lax.mul(lax.bitwise_and(v, 1), D))

        def fire_gather(b):
            pltpu.async_copy(w2_hbm.at[gidx_v[b]], rows_v[b], sem_g[b])

        def wait_gather(b):
            pltpu.make_async_copy(w2_hbm.at[gidx_v[b]], rows_v[b],
                                  sem_g[b]).wait()

        def transpose(b):
            zero = jnp.zeros((_L,), jnp.int32)
            iota = lax.iota(jnp.int32, _L)
            base = [pcol_v[b][pl.ds(g * _L, _L)] for g in range(_NG)]

            @plsc.parallel_loop(0, _L, 1, unroll=4)
            def _(j):
                rot = lax.bitwise_and(iota + j, _L - 1)
                for dblk in range(D // _L):
                    rv = rot + dblk * _L
                    sb = rv * _BB + iota
                    for g in range(_NG):
                        x = plsc.load_gather(rows_v[b], [zero, base[g] + rv])
                        plsc.store_scatter(out_v[b], [zero, sb + g * _L], x)

        def fire_writes(c, b):
            for r in range(D // 8):
                pltpu.async_copy(
                    out_v[b].at[pl.ds(r * 8, 8)],
                    out_hbm.at[c, pl.ds(r * 8, 8), pl.ds(col0, _BB)],
                    sem_w[b],
                )

        def wait_writes(b):
            for r in range(D // 8):
                pltpu.make_async_copy(
                    out_v[b].at[pl.ds(r * 8, 8)],
                    out_hbm.at[0, pl.ds(r * 8, 8), pl.ds(col0, _BB)],
                    sem_w[b],
                ).wait()

        load_idx(0, 0, True)
        prep_idx(0)
        fire_gather(0)
        load_idx(1, 1, True)
        prep_idx(1)
        fire_gather(1)
        load_idx(2, 2, False)

        GRPS = (S - 2) // NB

        def step(c, b, grp):
            b2 = (b + 2) % NB
            if grp is not None:
                wait_idx(b2)
                prep_idx(b2)
                fire_gather(b2)
                if b == 2:
                    @pl.when(grp < GRPS - 1)
                    def _():
                        load_idx(c + 3, b, False)
                else:
                    load_idx(c + 3, b, False)
            wait_gather(b)
            if grp is None:
                wait_writes(b)
            else:
                @pl.when(grp >= 1)
                def _():
                    wait_writes(b)
            transpose(b)
            fire_writes(c, b)

        def body(grp, carry):
            for b in range(NB):
                step(grp * NB + b, b, grp)
            return carry

        lax.fori_loop(0, GRPS, body, 0, unroll=False)
        step(S - 2, (S - 2) % NB, None)
        step(S - 1, (S - 1) % NB, None)
        for b in range(NB):
            wait_writes(b)

    return emb


def kernel(idx, weight):
    b, s = idx.shape
    v, d = weight.shape
    idx_t = idx.T.astype(jnp.int32)
    w2 = weight.reshape(v // 2, 2 * d)
    out_t = _emb_call(s, b, d)(idx_t, w2)
    return out_t.transpose(2, 0, 1)
